# baseline (speedup 1.0000x reference)
import functools

import jax
import jax.numpy as jnp
from jax import lax
from jax.experimental import pallas as pl
from jax.experimental.pallas import tpu as pltpu
from jax.experimental.pallas import tpu_sc as plsc

B = 1024
D = 64
HD = 3 * D
N_CONV_SEG = 50
N_TREE_SEG = 10
N_OCT = 70
OCT_PAD = 72
N_SEG = 60
SEG_PAD = 64
N_IDX = 8 * OCT_PAD


def _sc_segment_sums(idx, T):
    info = plsc.get_sparse_core_info()
    nc, ns = info.num_cores, info.num_subcores
    nw = nc * ns
    b_per_w = B // nw

    mesh = plsc.VectorSubcoreMesh(core_axis_name="c", subcore_axis_name="s")

    @functools.partial(
        pl.kernel,
        mesh=mesh,
        out_type=jax.ShapeDtypeStruct((B, SEG_PAD, HD), jnp.float32),
        scratch_types=[
            pltpu.VMEM((N_IDX,), jnp.int32),
            pltpu.VMEM((OCT_PAD, HD), jnp.float32),
            pltpu.VMEM((SEG_PAD - N_SEG, HD), jnp.float32),
            pltpu.SemaphoreType.DMA,
        ],
        compiler_params=pltpu.CompilerParams(use_tc_tiling_on_sc=False),
    )
    def sc_kernel(idx_hbm, t_hbm, out_hbm, idx_v, acc_v, zpad_v, sem):
        wid = lax.axis_index("s") * nc + lax.axis_index("c")
        zeros16 = jnp.zeros((16,), jnp.float32)
        for j in range(SEG_PAD - N_SEG):
            for v in range(HD // 16):
                zpad_v[j, pl.ds(v * 16, 16)] = zeros16

        def zero_acc(j, _):
            for v in range(HD // 16):
                acc_v[j, pl.ds(v * 16, 16)] = zeros16
            return 0

        def pair_sum(j, _):
            a = 50 + 2 * j
            for v in range(HD // 16):
                sl = pl.ds(v * 16, 16)
                acc_v[50 + j, sl] = acc_v[a, sl] + acc_v[a + 1, sl]
            return 0

        def do_row(i, _):
            b = wid * b_per_w + i
            pltpu.sync_copy(idx_hbm.at[b], idx_v)
            lax.fori_loop(0, OCT_PAD, zero_acc, 0)
            copies = [
                pltpu.async_copy(
                    t_hbm.at[idx_v.at[pl.ds(k * OCT_PAD, OCT_PAD)]],
                    acc_v,
                    sem,
                    add=True,
                )
                for k in range(8)
            ]
            for cp in copies:
                cp.wait()
            lax.fori_loop(0, N_TREE_SEG, pair_sum, 0)
            pltpu.sync_copy(acc_v.at[pl.ds(0, N_SEG)],
                            out_hbm.at[b, pl.ds(0, N_SEG)])
            pltpu.sync_copy(zpad_v, out_hbm.at[b, pl.ds(N_SEG, SEG_PAD - N_SEG)])
            return 0

        lax.fori_loop(0, b_per_w, do_row, 0)

    return sc_kernel(idx, T)


def _chain_body(s_ref, o_ref):
    S = s_ref[...]
    neg = jnp.float32(-1e30)
    jmask = lax.broadcasted_iota(jnp.int32, (1, SEG_PAD), 1) < N_SEG
    u = jnp.sum(S[:, :, 0:D], axis=1) * jnp.float32(1.0 / N_SEG)
    for h in (0, 1):
        sh = S[:, :, h * D : (h + 1) * D]
        sn = S[:, :, (h + 1) * D : (h + 2) * D]
        dots = jnp.sum(sh * u[:, None, :], axis=2)
        dots = jnp.where(jmask, dots, neg)
        m = jnp.max(dots, axis=1, keepdims=True)
        e = jnp.exp(dots - m)
        p = e / jnp.sum(e, axis=1, keepdims=True)
        u = u + jnp.sum(sn * p[:, :, None], axis=1)
    o_ref[...] = u


def _tc_chain(S):
    blk = 128
    return pl.pallas_call(
        _chain_body,
        grid=(B // blk,),
        in_specs=[pl.BlockSpec((blk, SEG_PAD, HD), lambda i: (i, 0, 0))],
        out_specs=pl.BlockSpec((blk, D), lambda i: (i, 0)),
        out_shape=jax.ShapeDtypeStruct((B, D), jnp.float32),
    )(S)


def kernel(conv_seqs, tree_tokens, C0, C1, C2, C3):
    conv_t = jnp.transpose(conv_seqs, (0, 2, 1))
    tree_t = jnp.transpose(
        tree_tokens.reshape(B, N_TREE_SEG, 2, 8), (0, 3, 1, 2)
    ).reshape(B, 8, 2 * N_TREE_SEG)
    idx = jnp.concatenate([conv_t, tree_t], axis=2)
    idx = jnp.pad(idx, ((0, 0), (0, 0), (0, OCT_PAD - N_OCT)))
    idx = idx.reshape(B, N_IDX)
    T = jnp.concatenate([C1, C2, C3], axis=1)
    S = _sc_segment_sums(idx, T)
    return _tc_chain(S)


# speedup vs baseline: 1.5597x; 1.5597x over previous
import functools

import jax
import jax.numpy as jnp
from jax import lax
from jax.experimental import pallas as pl
from jax.experimental.pallas import tpu as pltpu
from jax.experimental.pallas import tpu_sc as plsc

B = 1024
D = 64
HD = 3 * D
N_CONV = 400
N_TREE = 160
N_TOK = N_CONV + N_TREE
N_SEG = 60
CHUNK = 112
N_CHUNK = N_TOK // CHUNK
AHEAD = 3

_CHUNK_GROUPS = {
    0: [(14, 0, 0, 8)],
    1: [(14, 0, 14, 8)],
    2: [(14, 0, 28, 8)],
    3: [(8, 0, 42, 8), (3, 64, 50, 16)],
    4: [(7, 0, 53, 16)],
}


def _sc_segment_sums(idx, T):
    info = plsc.get_sparse_core_info()
    nc, ns = info.num_cores, info.num_subcores
    nw = nc * ns
    b_per_w = B // nw

    mesh = plsc.VectorSubcoreMesh(core_axis_name="c", subcore_axis_name="s")

    @functools.partial(
        pl.kernel,
        mesh=mesh,
        out_type=jax.ShapeDtypeStruct((B, N_SEG, HD), jnp.float32),
        scratch_types=[
            pltpu.VMEM((N_TOK,), jnp.int32),
            pltpu.VMEM((N_TOK,), jnp.int32),
            pltpu.VMEM((N_SEG, HD), jnp.float32),
            [pltpu.VMEM((CHUNK, HD), jnp.float32) for _ in range(N_CHUNK)],
            [pltpu.SemaphoreType.DMA for _ in range(N_CHUNK)],
        ],
        compiler_params=pltpu.CompilerParams(use_tc_tiling_on_sc=False),
    )
    def sc_kernel(idx_hbm, t_hbm, out_hbm, idx0, idx1, s_v, bufs, sems):
        wid = lax.axis_index("s") * nc + lax.axis_index("c")
        b0 = wid * b_per_w

        def fire(idx_v, c):
            pltpu.async_copy(
                t_hbm.at[idx_v.at[pl.ds(c * CHUNK, CHUNK)]],
                bufs[c],
                sems[c],
            )

        def drain(idx_v, c):
            pltpu.make_async_copy(
                t_hbm.at[idx_v.at[pl.ds(c * CHUNK, CHUNK)]],
                bufs[c],
                sems[c],
            ).wait()

        def reduce_group(c, n, row0, seg0, seg_len):
            buf = bufs[c]

            def body(j, _):
                base = row0 + j * seg_len
                seg = seg0 + j
                for v in range(HD // 16):
                    sl = pl.ds(v * 16, 16)
                    acc = buf[base, sl]
                    for k in range(1, seg_len):
                        acc = acc + buf[base + k, sl]
                    s_v[seg, sl] = acc
                return 0

            lax.fori_loop(0, n, body, 0)

        def do_row(i, idx_cur, idx_nxt):
            b = b0 + i
            pltpu.sync_copy(idx_hbm.at[jnp.minimum(b + 1, B - 1)], idx_nxt)
            for c in range(N_CHUNK):
                cn = c + AHEAD
                if cn < N_CHUNK:
                    fire(idx_cur, cn)
                else:
                    @pl.when(i + 1 < b_per_w)
                    def _():
                        fire(idx_nxt, cn - N_CHUNK)

                drain(idx_cur, c)
                for n, row0, seg0, seg_len in _CHUNK_GROUPS[c]:
                    reduce_group(c, n, row0, seg0, seg_len)
            pltpu.sync_copy(s_v, out_hbm.at[b])

        pltpu.sync_copy(idx_hbm.at[b0], idx0)
        for c in range(AHEAD):
            fire(idx0, c)

        def body2(i2, _):
            do_row(2 * i2, idx0, idx1)
            do_row(2 * i2 + 1, idx1, idx0)
            return 0

        lax.fori_loop(0, b_per_w // 2, body2, 0)

    return sc_kernel(idx, T)


def _chain_body(s_ref, o_ref):
    S = s_ref[...]
    u = jnp.sum(S[:, :, 0:D], axis=1) * jnp.float32(1.0 / N_SEG)
    for h in (0, 1):
        sh = S[:, :, h * D : (h + 1) * D]
        sn = S[:, :, (h + 1) * D : (h + 2) * D]
        dots = jnp.sum(sh * u[:, None, :], axis=2)
        m = jnp.max(dots, axis=1, keepdims=True)
        e = jnp.exp(dots - m)
        p = e / jnp.sum(e, axis=1, keepdims=True)
        u = u + jnp.sum(sn * p[:, :, None], axis=1)
    o_ref[...] = u


def _tc_chain(S):
    blk = 128
    return pl.pallas_call(
        _chain_body,
        grid=(B // blk,),
        in_specs=[pl.BlockSpec((blk, N_SEG, HD), lambda i: (i, 0, 0))],
        out_specs=pl.BlockSpec((blk, D), lambda i: (i, 0)),
        out_shape=jax.ShapeDtypeStruct((B, D), jnp.float32),
    )(S)


def kernel(conv_seqs, tree_tokens, C0, C1, C2, C3):
    idx = jnp.concatenate(
        [conv_seqs.reshape(B, N_CONV), tree_tokens.reshape(B, N_TREE)], axis=1
    )
    T = jnp.concatenate([C1, C2, C3], axis=1)
    S = _sc_segment_sums(idx, T)
    return _tc_chain(S)


# speedup vs baseline: 1.7609x; 1.1290x over previous
import functools

import jax
import jax.numpy as jnp
from jax import lax
from jax.experimental import pallas as pl
from jax.experimental.pallas import tpu as pltpu
from jax.experimental.pallas import tpu_sc as plsc

B = 1024
D = 64
HD = 3 * D
N_CONV = 400
N_TREE = 160
N_TOK = N_CONV + N_TREE
N_SEG = 60
CHUNK = 112
N_CHUNK = N_TOK // CHUNK
AHEAD = 3
NG = HD // 32

_CHUNK_GROUPS = {
    0: [(14, 0, 0, 8)],
    1: [(14, 0, 14, 8)],
    2: [(14, 0, 28, 8)],
    3: [(8, 0, 42, 8), (3, 64, 50, 16)],
    4: [(7, 0, 53, 16)],
}


def _sc_segment_sums(idx, T):
    info = plsc.get_sparse_core_info()
    nc, ns = info.num_cores, info.num_subcores
    nw = nc * ns
    b_per_w = B // nw

    mesh = plsc.VectorSubcoreMesh(core_axis_name="c", subcore_axis_name="s")

    @functools.partial(
        pl.kernel,
        mesh=mesh,
        out_type=jax.ShapeDtypeStruct((B, N_SEG, HD), jnp.bfloat16),
        scratch_types=[
            pltpu.VMEM((N_TOK,), jnp.int32),
            pltpu.VMEM((N_TOK,), jnp.int32),
            pltpu.VMEM((N_SEG, HD), jnp.bfloat16),
            [pltpu.VMEM((CHUNK, HD), jnp.bfloat16) for _ in range(N_CHUNK)],
            [pltpu.SemaphoreType.DMA for _ in range(N_CHUNK)],
        ],
        compiler_params=pltpu.CompilerParams(
            use_tc_tiling_on_sc=False, needs_layout_passes=False
        ),
    )
    def sc_kernel(idx_hbm, t_hbm, out_hbm, idx0, idx1, s_v, bufs, sems):
        wid = lax.axis_index("s") * nc + lax.axis_index("c")
        b0 = wid * b_per_w
        himask = jnp.full((16,), -65536, jnp.int32)

        def fire(idx_v, c):
            pltpu.async_copy(
                t_hbm.at[idx_v.at[pl.ds(c * CHUNK, CHUNK)]],
                bufs[c],
                sems[c],
            )

        def drain(idx_v, c):
            pltpu.make_async_copy(
                t_hbm.at[idx_v.at[pl.ds(c * CHUNK, CHUNK)]],
                bufs[c],
                sems[c],
            ).wait()

        def split(buf, row, g):
            w = plsc.bitcast(buf[row, pl.ds(g * 32, 32)], jnp.int32)
            lo = plsc.bitcast(w << 16, jnp.float32)
            hi = plsc.bitcast(w & himask, jnp.float32)
            return lo, hi

        def reduce_group(c, n, row0, seg0, seg_len):
            buf = bufs[c]

            def body(j, _):
                base = row0 + j * seg_len
                seg = seg0 + j
                for g in range(NG):
                    alo, ahi = split(buf, base, g)
                    for k in range(1, seg_len):
                        xlo, xhi = split(buf, base + k, g)
                        alo = alo + xlo
                        ahi = ahi + xhi
                    s_v[seg, pl.ds(g * 32, 32)] = plsc.pack(
                        alo, ahi, format=plsc.PackFormat.INTERLEAVED
                    )
                return 0

            lax.fori_loop(0, n, body, 0)

        def do_row(i, idx_cur, idx_nxt):
            b = b0 + i
            pltpu.sync_copy(idx_hbm.at[jnp.minimum(b + 1, B - 1)], idx_nxt)
            for c in range(N_CHUNK):
                cn = c + AHEAD
                if cn < N_CHUNK:
                    fire(idx_cur, cn)
                else:
                    @pl.when(i + 1 < b_per_w)
                    def _():
                        fire(idx_nxt, cn - N_CHUNK)

                drain(idx_cur, c)
                for n, row0, seg0, seg_len in _CHUNK_GROUPS[c]:
                    reduce_group(c, n, row0, seg0, seg_len)
            pltpu.sync_copy(s_v, out_hbm.at[b])

        pltpu.sync_copy(idx_hbm.at[b0], idx0)
        for c in range(AHEAD):
            fire(idx0, c)

        def body2(i2, _):
            do_row(2 * i2, idx0, idx1)
            do_row(2 * i2 + 1, idx1, idx0)
            return 0

        lax.fori_loop(0, b_per_w // 2, body2, 0)

    return sc_kernel(idx, T)


def _chain_body(s_ref, o_ref):
    S = s_ref[...].astype(jnp.float32)
    u = jnp.sum(S[:, :, 0:D], axis=1) * jnp.float32(1.0 / N_SEG)
    for h in (0, 1):
        sh = S[:, :, h * D : (h + 1) * D]
        sn = S[:, :, (h + 1) * D : (h + 2) * D]
        dots = jnp.sum(sh * u[:, None, :], axis=2)
        m = jnp.max(dots, axis=1, keepdims=True)
        e = jnp.exp(dots - m)
        p = e / jnp.sum(e, axis=1, keepdims=True)
        u = u + jnp.sum(sn * p[:, :, None], axis=1)
    o_ref[...] = u


def _tc_chain(S):
    blk = 128
    return pl.pallas_call(
        _chain_body,
        grid=(B // blk,),
        in_specs=[pl.BlockSpec((blk, N_SEG, HD), lambda i: (i, 0, 0))],
        out_specs=pl.BlockSpec((blk, D), lambda i: (i, 0)),
        out_shape=jax.ShapeDtypeStruct((B, D), jnp.float32),
    )(S)


def kernel(conv_seqs, tree_tokens, C0, C1, C2, C3):
    idx = jnp.concatenate(
        [conv_seqs.reshape(B, N_CONV), tree_tokens.reshape(B, N_TREE)], axis=1
    )
    T = jnp.concatenate([C1, C2, C3], axis=1).astype(jnp.bfloat16)
    S = _sc_segment_sums(idx, T)
    return _tc_chain(S)


# speedup vs baseline: 1.8374x; 1.0434x over previous
import functools

import jax
import jax.numpy as jnp
from jax import lax
from jax.experimental import pallas as pl
from jax.experimental.pallas import tpu as pltpu
from jax.experimental.pallas import tpu_sc as plsc

B = 1024
D = 64
HD = 3 * D
N_CONV = 400
N_TREE = 160
N_TOK = N_CONV + N_TREE
N_SEG = 60
SEG_PAD = 64
HD_PAD = 256
CHUNK = 112
N_CHUNK = N_TOK // CHUNK
AHEAD = 3
NG = HD // 32

_CHUNK_GROUPS = {
    0: [(14, 0, 0, 8)],
    1: [(14, 0, 14, 8)],
    2: [(14, 0, 28, 8)],
    3: [(8, 0, 42, 8), (3, 64, 50, 16)],
    4: [(7, 0, 53, 16)],
}


def _sc_segment_sums(conv, tree, T):
    info = plsc.get_sparse_core_info()
    nc, ns = info.num_cores, info.num_subcores
    nw = nc * ns
    b_per_w = B // nw

    mesh = plsc.VectorSubcoreMesh(core_axis_name="c", subcore_axis_name="s")

    @functools.partial(
        pl.kernel,
        mesh=mesh,
        out_type=jax.ShapeDtypeStruct((B, SEG_PAD, HD_PAD), jnp.bfloat16),
        scratch_types=[
            pltpu.VMEM((N_TOK,), jnp.int32),
            pltpu.VMEM((N_TOK,), jnp.int32),
            pltpu.VMEM((SEG_PAD, HD_PAD), jnp.bfloat16),
            [pltpu.VMEM((CHUNK, HD), jnp.bfloat16) for _ in range(N_CHUNK)],
            [pltpu.SemaphoreType.DMA for _ in range(N_CHUNK)],
        ],
        compiler_params=pltpu.CompilerParams(
            use_tc_tiling_on_sc=False, needs_layout_passes=False
        ),
    )
    def sc_kernel(conv_hbm, tree_hbm, t_hbm, out_hbm, idx0, idx1, s_v, bufs,
                  sems):
        wid = lax.axis_index("s") * nc + lax.axis_index("c")
        b0 = wid * b_per_w
        himask = jnp.full((16,), -65536, jnp.int32)
        zero32 = jnp.zeros((32,), jnp.bfloat16)
        for j in range(SEG_PAD):
            start = 0 if j >= N_SEG else HD
            for g in range(start, HD_PAD, 32):
                s_v[j, pl.ds(g, 32)] = zero32

        def fire(idx_v, c):
            pltpu.async_copy(
                t_hbm.at[idx_v.at[pl.ds(c * CHUNK, CHUNK)]],
                bufs[c],
                sems[c],
            )

        def drain(idx_v, c):
            pltpu.make_async_copy(
                t_hbm.at[idx_v.at[pl.ds(c * CHUNK, CHUNK)]],
                bufs[c],
                sems[c],
            ).wait()

        def split(buf, row, g):
            w = plsc.bitcast(buf[row, pl.ds(g * 32, 32)], jnp.int32)
            lo = plsc.bitcast(w << 16, jnp.float32)
            hi = plsc.bitcast(w & himask, jnp.float32)
            return lo, hi

        def reduce_group(c, n, row0, seg0, seg_len):
            buf = bufs[c]

            def body(j, _):
                base = row0 + j * seg_len
                seg = seg0 + j
                for g in range(NG):
                    alo, ahi = split(buf, base, g)
                    for k in range(1, seg_len):
                        xlo, xhi = split(buf, base + k, g)
                        alo = alo + xlo
                        ahi = ahi + xhi
                    s_v[seg, pl.ds(g * 32, 32)] = plsc.pack(
                        alo, ahi, format=plsc.PackFormat.INTERLEAVED
                    )
                return 0

            lax.fori_loop(0, n, body, 0)

        def do_row(i, idx_cur, idx_nxt):
            b = b0 + i
            bn = jnp.minimum(b + 1, B - 1)
            pltpu.sync_copy(conv_hbm.at[bn], idx_nxt.at[pl.ds(0, N_CONV)])
            pltpu.sync_copy(tree_hbm.at[bn], idx_nxt.at[pl.ds(N_CONV, N_TREE)])
            for c in range(N_CHUNK):
                cn = c + AHEAD
                if cn < N_CHUNK:
                    fire(idx_cur, cn)
                else:
                    @pl.when(i + 1 < b_per_w)
                    def _():
                        fire(idx_nxt, cn - N_CHUNK)

                drain(idx_cur, c)
                for n, row0, seg0, seg_len in _CHUNK_GROUPS[c]:
                    reduce_group(c, n, row0, seg0, seg_len)
            pltpu.sync_copy(s_v, out_hbm.at[b])

        pltpu.sync_copy(conv_hbm.at[b0], idx0.at[pl.ds(0, N_CONV)])
        pltpu.sync_copy(tree_hbm.at[b0], idx0.at[pl.ds(N_CONV, N_TREE)])
        for c in range(AHEAD):
            fire(idx0, c)

        def body2(i2, _):
            do_row(2 * i2, idx0, idx1)
            do_row(2 * i2 + 1, idx1, idx0)
            return 0

        lax.fori_loop(0, b_per_w // 2, body2, 0)

    return sc_kernel(conv, tree, T)


def _chain_body(s_ref, o_ref):
    S = s_ref[...].astype(jnp.float32)
    neg = jnp.float32(-1e30)
    jmask = lax.broadcasted_iota(jnp.int32, (1, SEG_PAD), 1) < N_SEG
    u = jnp.sum(S[:, :, 0:D], axis=1) * jnp.float32(1.0 / N_SEG)
    for h in (0, 1):
        sh = S[:, :, h * D : (h + 1) * D]
        sn = S[:, :, (h + 1) * D : (h + 2) * D]
        dots = jnp.sum(sh * u[:, None, :], axis=2)
        dots = jnp.where(jmask, dots, neg)
        m = jnp.max(dots, axis=1, keepdims=True)
        e = jnp.exp(dots - m)
        p = e / jnp.sum(e, axis=1, keepdims=True)
        u = u + jnp.sum(sn * p[:, :, None], axis=1)
    o_ref[...] = u


def _tc_chain(S):
    blk = 128
    return pl.pallas_call(
        _chain_body,
        grid=(B // blk,),
        in_specs=[pl.BlockSpec((blk, SEG_PAD, HD_PAD), lambda i: (i, 0, 0))],
        out_specs=pl.BlockSpec((blk, D), lambda i: (i, 0)),
        out_shape=jax.ShapeDtypeStruct((B, D), jnp.float32),
    )(S)


def kernel(conv_seqs, tree_tokens, C0, C1, C2, C3):
    T = jnp.concatenate([C1, C2, C3], axis=1).astype(jnp.bfloat16)
    S = _sc_segment_sums(
        conv_seqs.reshape(B, N_CONV), tree_tokens.reshape(B, N_TREE), T
    )
    return _tc_chain(S)


# speedup vs baseline: 1.8631x; 1.0140x over previous
import functools

import jax
import jax.numpy as jnp
from jax import lax
from jax.experimental import pallas as pl
from jax.experimental.pallas import tpu as pltpu
from jax.experimental.pallas import tpu_sc as plsc

B = 1024
D = 64
HD = 3 * D
N_CONV = 400
N_TREE = 160
N_TOK = N_CONV + N_TREE
N_SEG = 60
SEG_PAD = 64
HD_PAD = 256
CHUNK = 112
N_CHUNK = N_TOK // CHUNK
AHEAD = 3
NG = HD // 32

_CHUNK_GROUPS = {
    0: [(14, 0, 0, 8)],
    1: [(14, 0, 14, 8)],
    2: [(14, 0, 28, 8)],
    3: [(8, 0, 42, 8), (3, 64, 50, 16)],
    4: [(7, 0, 53, 16)],
}


def _sc_segment_sums(conv, tree, T):
    info = plsc.get_sparse_core_info()
    nc, ns = info.num_cores, info.num_subcores
    nw = nc * ns
    b_per_w = B // nw

    mesh = plsc.VectorSubcoreMesh(core_axis_name="c", subcore_axis_name="s")

    @functools.partial(
        pl.kernel,
        mesh=mesh,
        out_type=jax.ShapeDtypeStruct((B, SEG_PAD, HD_PAD), jnp.bfloat16),
        scratch_types=[
            pltpu.VMEM((N_TOK,), jnp.int32),
            pltpu.VMEM((N_TOK,), jnp.int32),
            pltpu.VMEM((SEG_PAD, HD_PAD), jnp.bfloat16),
            pltpu.VMEM((SEG_PAD, HD_PAD), jnp.bfloat16),
            [pltpu.VMEM((CHUNK, HD), jnp.bfloat16) for _ in range(N_CHUNK)],
            [pltpu.SemaphoreType.DMA for _ in range(N_CHUNK)],
            pltpu.SemaphoreType.DMA,
            pltpu.SemaphoreType.DMA,
        ],
        compiler_params=pltpu.CompilerParams(
            use_tc_tiling_on_sc=False, needs_layout_passes=False
        ),
    )
    def sc_kernel(conv_hbm, tree_hbm, t_hbm, out_hbm, idx0, idx1, s_v0, s_v1,
                  bufs, sems, osem0, osem1):
        wid = lax.axis_index("s") * nc + lax.axis_index("c")
        b0 = wid * b_per_w
        zero32 = jnp.zeros((32,), jnp.bfloat16)
        for s_v in (s_v0, s_v1):
            for j in range(SEG_PAD):
                start = 0 if j >= N_SEG else HD
                for g in range(start, HD_PAD, 32):
                    s_v[j, pl.ds(g, 32)] = zero32

        def fire(idx_v, c):
            pltpu.async_copy(
                t_hbm.at[idx_v.at[pl.ds(c * CHUNK, CHUNK)]],
                bufs[c],
                sems[c],
            )

        def drain(idx_v, c):
            pltpu.make_async_copy(
                t_hbm.at[idx_v.at[pl.ds(c * CHUNK, CHUNK)]],
                bufs[c],
                sems[c],
            ).wait()

        def split(buf, row, g):
            w = plsc.bitcast(buf[row, pl.ds(g * 32, 32)], jnp.int32)
            lo = plsc.bitcast(w << 16, jnp.float32)
            hi = plsc.bitcast(w, jnp.float32)
            return lo, hi

        def reduce_group(s_v, c, n, row0, seg0, seg_len):
            buf = bufs[c]

            def body(j, _):
                base = row0 + j * seg_len
                seg = seg0 + j
                for g in range(NG):
                    alo, ahi = split(buf, base, g)
                    for k in range(1, seg_len):
                        xlo, xhi = split(buf, base + k, g)
                        alo = alo + xlo
                        ahi = ahi + xhi
                    s_v[seg, pl.ds(g * 32, 32)] = plsc.pack(
                        alo, ahi, format=plsc.PackFormat.INTERLEAVED
                    )
                return 0

            lax.fori_loop(0, n, body, 0)

        def do_row(i, idx_cur, idx_nxt, s_v, osem):
            b = b0 + i
            bn = jnp.minimum(b + 1, B - 1)
            pltpu.sync_copy(conv_hbm.at[bn], idx_nxt.at[pl.ds(0, N_CONV)])
            pltpu.sync_copy(tree_hbm.at[bn], idx_nxt.at[pl.ds(N_CONV, N_TREE)])
            for c in range(N_CHUNK):
                cn = c + AHEAD
                if cn < N_CHUNK:
                    fire(idx_cur, cn)
                else:
                    @pl.when(i + 1 < b_per_w)
                    def _():
                        fire(idx_nxt, cn - N_CHUNK)

                drain(idx_cur, c)
                if c == 0:
                    @pl.when(i >= 2)
                    def _():
                        pltpu.make_async_copy(
                            s_v, out_hbm.at[jnp.maximum(b - 2, 0)], osem
                        ).wait()
                for n, row0, seg0, seg_len in _CHUNK_GROUPS[c]:
                    reduce_group(s_v, c, n, row0, seg0, seg_len)
            pltpu.async_copy(s_v, out_hbm.at[b], osem)

        pltpu.sync_copy(conv_hbm.at[b0], idx0.at[pl.ds(0, N_CONV)])
        pltpu.sync_copy(tree_hbm.at[b0], idx0.at[pl.ds(N_CONV, N_TREE)])
        for c in range(AHEAD):
            fire(idx0, c)

        def body2(i2, _):
            do_row(2 * i2, idx0, idx1, s_v0, osem0)
            do_row(2 * i2 + 1, idx1, idx0, s_v1, osem1)
            return 0

        lax.fori_loop(0, b_per_w // 2, body2, 0)
        pltpu.make_async_copy(s_v0, out_hbm.at[b0 + b_per_w - 2], osem0).wait()
        pltpu.make_async_copy(s_v1, out_hbm.at[b0 + b_per_w - 1], osem1).wait()

    return sc_kernel(conv, tree, T)


def _chain_body(s_ref, o_ref):
    S = s_ref[...].astype(jnp.float32)
    neg = jnp.float32(-1e30)
    jmask = lax.broadcasted_iota(jnp.int32, (1, SEG_PAD), 1) < N_SEG
    u = jnp.sum(S[:, :, 0:D], axis=1) * jnp.float32(1.0 / N_SEG)
    for h in (0, 1):
        sh = S[:, :, h * D : (h + 1) * D]
        sn = S[:, :, (h + 1) * D : (h + 2) * D]
        dots = jnp.sum(sh * u[:, None, :], axis=2)
        dots = jnp.where(jmask, dots, neg)
        m = jnp.max(dots, axis=1, keepdims=True)
        e = jnp.exp(dots - m)
        p = e / jnp.sum(e, axis=1, keepdims=True)
        u = u + jnp.sum(sn * p[:, :, None], axis=1)
    o_ref[...] = u


def _tc_chain(S):
    blk = 256
    return pl.pallas_call(
        _chain_body,
        grid=(B // blk,),
        in_specs=[pl.BlockSpec((blk, SEG_PAD, HD_PAD), lambda i: (i, 0, 0))],
        out_specs=pl.BlockSpec((blk, D), lambda i: (i, 0)),
        out_shape=jax.ShapeDtypeStruct((B, D), jnp.float32),
    )(S)


def kernel(conv_seqs, tree_tokens, C0, C1, C2, C3):
    T = jnp.concatenate([C1, C2, C3], axis=1).astype(jnp.bfloat16)
    S = _sc_segment_sums(
        conv_seqs.reshape(B, N_CONV), tree_tokens.reshape(B, N_TREE), T
    )
    return _tc_chain(S)
